# baseline (speedup 1.0000x reference)
import functools

import jax
import jax.numpy as jnp
from jax import lax
from jax.experimental import pallas as pl
from jax.experimental.pallas import tpu as pltpu
from jax.experimental.pallas import tpu_sc as plsc

_D = 64
_B = 16384
_NC = 2
_NS = 16
_NW = _NC * _NS
_BPW = _B // _NW
_NCHUNK = 4
_CHUNK = _BPW // _NCHUNK
_GRP = 16
_GROUPS = _CHUNK // _GRP

_mesh = plsc.VectorSubcoreMesh(core_axis_name="c", subcore_axis_name="s")


def _gmf_body(uidx_hbm, iidx_hbm, utab_hbm, itab_hbm, w_hbm, b_hbm, out_hbm,
              uidx_v, iidx_v, urows_v, irows_v, w_v, b_v, pacc_v, out_v,
              *sems):
    wid = lax.axis_index("s") * _NC + lax.axis_index("c")
    base = wid * _BPW

    pltpu.sync_copy(uidx_hbm.at[wid], uidx_v)
    pltpu.sync_copy(iidx_hbm.at[wid], iidx_v)
    pltpu.sync_copy(w_hbm, w_v)
    pltpu.sync_copy(b_hbm, b_v)

    copies = []
    for c in range(_NCHUNK):
        cu = pltpu.async_copy(utab_hbm.at[uidx_v.at[c]],
                              urows_v.at[pl.ds(c * _CHUNK, _CHUNK)],
                              sems[2 * c])
        ci = pltpu.async_copy(itab_hbm.at[iidx_v.at[c]],
                              irows_v.at[pl.ds(c * _CHUNK, _CHUNK)],
                              sems[2 * c + 1])
        copies.append((cu, ci))

    w0 = w_v[pl.ds(0, 16)]
    w1 = w_v[pl.ds(16, 16)]
    w2 = w_v[pl.ds(32, 16)]
    w3 = w_v[pl.ds(48, 16)]
    bvec = b_v[...]
    lane = lax.iota(jnp.int32, 16)

    def make_group_body(c):
        def group_body(g, _):
            r0 = c * _CHUNK + g * _GRP
            for j in range(_GRP):
                r = r0 + j
                s = (urows_v[r, pl.ds(0, 16)] * irows_v[r, pl.ds(0, 16)] * w0
                     + urows_v[r, pl.ds(16, 16)] * irows_v[r, pl.ds(16, 16)] * w1
                     + urows_v[r, pl.ds(32, 16)] * irows_v[r, pl.ds(32, 16)] * w2
                     + urows_v[r, pl.ds(48, 16)] * irows_v[r, pl.ds(48, 16)] * w3)
                pacc_v[j] = s
            acc = plsc.load_gather(pacc_v, [lane, jnp.zeros((16,), jnp.int32)])
            for col in range(1, 16):
                acc = acc + plsc.load_gather(
                    pacc_v, [lane, jnp.full((16,), col, jnp.int32)])
            logits = acc + bvec
            rating = 1.0 / (1.0 + jnp.exp(-logits))
            out_v[pl.ds(r0, _GRP)] = rating
            return _
        return group_body

    for c in range(_NCHUNK):
        cu, ci = copies[c]
        cu.wait()
        ci.wait()
        lax.fori_loop(0, _GROUPS, make_group_body(c), None)

    pltpu.sync_copy(out_v, out_hbm.at[pl.ds(base, _BPW)])


_gmf = functools.partial(
    pl.kernel,
    mesh=_mesh,
    compiler_params=pltpu.CompilerParams(
        needs_layout_passes=False, use_tc_tiling_on_sc=False),
    out_type=jax.ShapeDtypeStruct((_B,), jnp.float32),
    scratch_types=[
        pltpu.VMEM((_NCHUNK, _CHUNK), jnp.int32),
        pltpu.VMEM((_NCHUNK, _CHUNK), jnp.int32),
        pltpu.VMEM((_BPW, _D), jnp.float32),
        pltpu.VMEM((_BPW, _D), jnp.float32),
        pltpu.VMEM((_D,), jnp.float32),
        pltpu.VMEM((16,), jnp.float32),
        pltpu.VMEM((_GRP, 16), jnp.float32),
        pltpu.VMEM((_BPW,), jnp.float32),
    ] + [pltpu.SemaphoreType.DMA] * (2 * _NCHUNK),
)(_gmf_body)


@jax.jit
def kernel(user_indices, item_indices, embedding_user, embedding_item,
           affine_w, affine_b):
    uidx = user_indices.astype(jnp.int32).reshape(_NW, _NCHUNK, _CHUNK)
    iidx = item_indices.astype(jnp.int32).reshape(_NW, _NCHUNK, _CHUNK)
    w = affine_w.astype(jnp.float32).reshape(_D)
    b = jnp.broadcast_to(affine_b.astype(jnp.float32).reshape(1), (16,))
    out = _gmf(uidx, iidx, embedding_user, embedding_item, w, b)
    return out.reshape(_B, 1)


# speedup vs baseline: 1.3481x; 1.3481x over previous
import functools

import jax
import jax.numpy as jnp
from jax import lax
from jax.experimental import pallas as pl
from jax.experimental.pallas import tpu as pltpu
from jax.experimental.pallas import tpu_sc as plsc

_D = 64
_B = 16384
_NC = 2
_NS = 16
_NW = _NC * _NS
_BPW = _B // _NW
_NCHUNK = 4
_CHUNK = _BPW // _NCHUNK
_GRP = 16
_GROUPS = _CHUNK // _GRP

_mesh = plsc.VectorSubcoreMesh(core_axis_name="c", subcore_axis_name="s")


def _gmf_body(uidx_hbm, iidx_hbm, utab_hbm, itab_hbm, w_hbm, b_hbm, out_hbm,
              uidx_v, iidx_v, urows_v, irows_v, w_v, b_v, pacc_v, out_v,
              su0, su1, si0, si1):
    wid = lax.axis_index("s") * _NC + lax.axis_index("c")
    base = wid * _BPW
    sem_u = (su0, su1)
    sem_i = (si0, si1)

    pltpu.sync_copy(uidx_hbm.at[pl.ds(base, _BPW)], uidx_v)
    pltpu.sync_copy(iidx_hbm.at[pl.ds(base, _BPW)], iidx_v)
    pltpu.sync_copy(w_hbm, w_v)
    pltpu.sync_copy(b_hbm, b_v)

    def enqueue_chunk(c):
        p = c % 2

        def body(g, _):
            vu = uidx_v[pl.ds(c * _CHUNK + g * _GRP, _GRP)]
            vi = iidx_v[pl.ds(c * _CHUNK + g * _GRP, _GRP)]
            for j in range(_GRP):
                lr = g * _GRP + j
                pltpu.async_copy(utab_hbm.at[vu[j]], urows_v.at[p, lr],
                                 sem_u[p])
                pltpu.async_copy(itab_hbm.at[vi[j]], irows_v.at[p, lr],
                                 sem_i[p])
            return _

        lax.fori_loop(0, _GROUPS, body, None)

    def wait_chunk(c):
        p = c % 2
        pltpu.make_async_copy(utab_hbm.at[pl.ds(0, _CHUNK)],
                              urows_v.at[p], sem_u[p]).wait()
        pltpu.make_async_copy(itab_hbm.at[pl.ds(0, _CHUNK)],
                              irows_v.at[p], sem_i[p]).wait()

    w0 = w_v[pl.ds(0, 16)]
    w1 = w_v[pl.ds(16, 16)]
    w2 = w_v[pl.ds(32, 16)]
    w3 = w_v[pl.ds(48, 16)]
    bvec = b_v[...]
    lane = lax.iota(jnp.int32, 16)

    def make_group_body(c):
        p = c % 2

        def group_body(g, _):
            for j in range(_GRP):
                lr = g * _GRP + j
                s = (urows_v[p, lr, pl.ds(0, 16)]
                     * irows_v[p, lr, pl.ds(0, 16)] * w0
                     + urows_v[p, lr, pl.ds(16, 16)]
                     * irows_v[p, lr, pl.ds(16, 16)] * w1
                     + urows_v[p, lr, pl.ds(32, 16)]
                     * irows_v[p, lr, pl.ds(32, 16)] * w2
                     + urows_v[p, lr, pl.ds(48, 16)]
                     * irows_v[p, lr, pl.ds(48, 16)] * w3)
                pacc_v[j] = s
            acc = plsc.load_gather(pacc_v, [lane, jnp.zeros((16,), jnp.int32)])
            for col in range(1, 16):
                acc = acc + plsc.load_gather(
                    pacc_v, [lane, jnp.full((16,), col, jnp.int32)])
            logits = acc + bvec
            rating = 1.0 / (1.0 + jnp.exp(-logits))
            out_v[pl.ds(c * _CHUNK + g * _GRP, _GRP)] = rating
            return _

        return group_body

    enqueue_chunk(0)
    for c in range(_NCHUNK):
        if c + 1 < _NCHUNK:
            enqueue_chunk(c + 1)
        wait_chunk(c)
        lax.fori_loop(0, _GROUPS, make_group_body(c), None)

    pltpu.sync_copy(out_v, out_hbm.at[pl.ds(base, _BPW)])


_gmf = functools.partial(
    pl.kernel,
    mesh=_mesh,
    compiler_params=pltpu.CompilerParams(needs_layout_passes=False),
    out_type=jax.ShapeDtypeStruct((_B,), jnp.float32),
    scratch_types=[
        pltpu.VMEM((_BPW,), jnp.int32),
        pltpu.VMEM((_BPW,), jnp.int32),
        pltpu.VMEM((2, _CHUNK, _D), jnp.float32),
        pltpu.VMEM((2, _CHUNK, _D), jnp.float32),
        pltpu.VMEM((_D,), jnp.float32),
        pltpu.VMEM((16,), jnp.float32),
        pltpu.VMEM((_GRP, 16), jnp.float32),
        pltpu.VMEM((_BPW,), jnp.float32),
    ] + [pltpu.SemaphoreType.DMA] * 4,
)(_gmf_body)


@jax.jit
def kernel(user_indices, item_indices, embedding_user, embedding_item,
           affine_w, affine_b):
    uidx = user_indices.astype(jnp.int32)
    iidx = item_indices.astype(jnp.int32)
    w = affine_w.astype(jnp.float32).reshape(_D)
    b = jnp.broadcast_to(affine_b.astype(jnp.float32).reshape(1), (16,))
    out = _gmf(uidx, iidx, embedding_user, embedding_item, w, b)
    return out.reshape(_B, 1)
